# Initial kernel scaffold; baseline (speedup 1.0000x reference)
import jax
import jax.numpy as jnp
from jax.experimental import pallas as pl


def kernel(in0, in1, in2):
    raise NotImplementedError("write your pallas kernel here")


# speedup vs baseline: 2.4001x; 2.4001x over previous
import jax
import jax.numpy as jnp
from jax import lax
from jax.experimental import pallas as pl
from jax.experimental.pallas import tpu as pltpu
from jax.experimental.pallas import tpu_sc as plsc

BLK = 40
NW = 32
LANES = 16


def _row_compute(in0_v, in1_v, in2_v, out_v, nrows):
    def row(r, carry):
        a0 = in0_v[r, pl.ds(0, 16)]
        a1 = in0_v[r, pl.ds(16, 16)]
        a2 = in0_v[r, pl.ds(32, 16)]
        a3 = in0_v[r, pl.ds(48, 16)]
        b0 = in1_v[r, pl.ds(0, 16)]
        b1 = in1_v[r, pl.ds(16, 16)]
        b2 = in1_v[r, pl.ds(32, 16)]
        b3 = in1_v[r, pl.ds(48, 16)]
        p00 = a0 * (0.5 * b0 + 0.25 * b2)
        p01 = a1 * (0.5 * b1 + 0.25 * b3)
        p10 = a2 * (0.75 * b0 - 0.25 * b2)
        p11 = a3 * (0.75 * b1 - 0.25 * b3)

        acc0 = [None] * 4
        acc1 = [None] * 4
        for u in range(32):
            plo = p00[u] if u < 16 else p01[u - 16]
            phi = p10[u] if u < 16 else p11[u - 16]
            v0 = in2_v[r, pl.ds(u * 16, 16)] * plo
            v1 = in2_v[r, pl.ds(512 + u * 16, 16)] * phi
            j = u % 4
            if u < 4:
                acc0[j] = v0
                acc1[j] = v1
            else:
                acc0[j] = acc0[j] + v0
                acc1[j] = acc1[j] + v1
        out_v[r, pl.ds(0, 16)] = (acc0[0] + acc0[1]) + (acc0[2] + acc0[3])
        out_v[r, pl.ds(16, 16)] = (acc1[0] + acc1[1]) + (acc1[2] + acc1[3])
        return carry

    lax.fori_loop(0, nrows, row, 0)


def _make_tec_body(T, EXTRA):
    def tec_body(in0_h, in1_h, in2_h, out_h,
                 i0a, i1a, i2a, oa, i0b, i1b, i2b, ob,
                 sia, sib, soa, sob):
        cid = lax.axis_index("c")
        sid = lax.axis_index("s")
        wid = sid * 2 + cid

        def start_in(t, d0, d1, d2, sem):
            r0 = (t * NW + wid) * BLK
            pltpu.make_async_copy(in0_h.at[pl.ds(r0, BLK), :], d0, sem).start()
            pltpu.make_async_copy(in1_h.at[pl.ds(r0, BLK), :], d1, sem).start()
            pltpu.make_async_copy(in2_h.at[pl.ds(r0, BLK), :], d2, sem).start()

        def wait_in(d0, d1, d2, sem):
            pltpu.make_async_copy(in0_h.at[pl.ds(0, BLK), :], d0, sem).wait()
            pltpu.make_async_copy(in1_h.at[pl.ds(0, BLK), :], d1, sem).wait()
            pltpu.make_async_copy(in2_h.at[pl.ds(0, BLK), :], d2, sem).wait()

        def start_out(t, src, sem):
            r0 = (t * NW + wid) * BLK
            pltpu.make_async_copy(src, out_h.at[pl.ds(r0, BLK), :], sem).start()

        def wait_out(src, sem):
            pltpu.make_async_copy(src, out_h.at[pl.ds(0, BLK), :], sem).wait()

        start_in(0, i0a, i1a, i2a, sia)

        def pair(i, carry):
            t0 = 2 * i
            wait_in(i0a, i1a, i2a, sia)
            start_in(t0 + 1, i0b, i1b, i2b, sib)

            @pl.when(i > 0)
            def _():
                wait_out(oa, soa)

            _row_compute(i0a, i1a, i2a, oa, BLK)
            start_out(t0, oa, soa)

            wait_in(i0b, i1b, i2b, sib)

            @pl.when(t0 + 2 < T)
            def _():
                start_in(t0 + 2, i0a, i1a, i2a, sia)

            @pl.when(i > 0)
            def _():
                wait_out(ob, sob)

            _row_compute(i0b, i1b, i2b, ob, BLK)
            start_out(t0 + 1, ob, sob)
            return carry

        lax.fori_loop(0, T // 2, pair, 0)
        wait_out(oa, soa)
        wait_out(ob, sob)

        for e in range(EXTRA):
            @pl.when(wid == (e % NW))
            def _():
                r0 = (T * NW + e) * BLK
                pltpu.sync_copy(in0_h.at[pl.ds(r0, BLK), :], i0a)
                pltpu.sync_copy(in1_h.at[pl.ds(r0, BLK), :], i1a)
                pltpu.sync_copy(in2_h.at[pl.ds(r0, BLK), :], i2a)
                _row_compute(i0a, i1a, i2a, oa, BLK)
                pltpu.sync_copy(oa, out_h.at[pl.ds(r0, BLK), :])

    return tec_body


def _build(N):
    assert N % BLK == 0, N
    nblk = N // BLK
    T = (nblk // NW) & ~1
    EXTRA = nblk - T * NW
    f32 = jnp.float32
    mesh = plsc.VectorSubcoreMesh(
        core_axis_name="c", subcore_axis_name="s", num_cores=2, num_subcores=16
    )
    return pl.kernel(
        _make_tec_body(T, EXTRA),
        out_type=jax.ShapeDtypeStruct((N, 32), f32),
        mesh=mesh,
        scratch_types=[
            pltpu.VMEM((BLK, 64), f32),
            pltpu.VMEM((BLK, 64), f32),
            pltpu.VMEM((BLK, 1024), f32),
            pltpu.VMEM((BLK, 32), f32),
            pltpu.VMEM((BLK, 64), f32),
            pltpu.VMEM((BLK, 64), f32),
            pltpu.VMEM((BLK, 1024), f32),
            pltpu.VMEM((BLK, 32), f32),
            pltpu.SemaphoreType.DMA,
            pltpu.SemaphoreType.DMA,
            pltpu.SemaphoreType.DMA,
            pltpu.SemaphoreType.DMA,
        ],
    )


def kernel(in0, in1, in2):
    return _build(in0.shape[0])(in0, in1, in2)


# speedup vs baseline: 2.6533x; 1.1055x over previous
import jax
import jax.numpy as jnp
from jax import lax
from jax.experimental import pallas as pl
from jax.experimental.pallas import tpu as pltpu
from jax.experimental.pallas import tpu_sc as plsc

BLK = 40
NW = 32
PB = 2048


def _p_block(in0t_ref, in1t_ref, p_ref):
    a = in0t_ref[...]
    b = in1t_ref[...]
    plo = a[0:32, :] * (0.5 * b[0:32, :] + 0.25 * b[32:64, :])
    phi = a[32:64, :] * (0.75 * b[0:32, :] - 0.25 * b[32:64, :])
    pt = jnp.concatenate([plo, phi], axis=0)
    p_ref[...] = pt.T


def _p_kernel(in0t, in1t):
    n = in0t.shape[1]
    grid = (n + PB - 1) // PB
    return pl.pallas_call(
        _p_block,
        grid=(grid,),
        in_specs=[
            pl.BlockSpec((64, PB), lambda i: (0, i)),
            pl.BlockSpec((64, PB), lambda i: (0, i)),
        ],
        out_specs=pl.BlockSpec((PB, 64), lambda i: (i, 0)),
        out_shape=jax.ShapeDtypeStruct((n, 64), jnp.float32),
    )(in0t, in1t)


def _row_compute(p_v, in2_v, out_v, nrows):
    def row(r, carry):
        p00 = p_v[r, pl.ds(0, 16)]
        p01 = p_v[r, pl.ds(16, 16)]
        p10 = p_v[r, pl.ds(32, 16)]
        p11 = p_v[r, pl.ds(48, 16)]

        acc0 = [None] * 4
        acc1 = [None] * 4
        for u in range(32):
            plo = p00[u] if u < 16 else p01[u - 16]
            phi = p10[u] if u < 16 else p11[u - 16]
            v0 = in2_v[r, pl.ds(u * 16, 16)] * plo
            v1 = in2_v[r, pl.ds(512 + u * 16, 16)] * phi
            j = u % 4
            if u < 4:
                acc0[j] = v0
                acc1[j] = v1
            else:
                acc0[j] = acc0[j] + v0
                acc1[j] = acc1[j] + v1
        out_v[r, pl.ds(0, 16)] = (acc0[0] + acc0[1]) + (acc0[2] + acc0[3])
        out_v[r, pl.ds(16, 16)] = (acc1[0] + acc1[1]) + (acc1[2] + acc1[3])
        return carry

    lax.fori_loop(0, nrows, row, 0)


def _make_tec_body(T, EXTRA):
    def tec_body(p_h, in2_h, out_h,
                 pA, mA, oA, pB, mB, oB,
                 sia, sib, soa, sob):
        cid = lax.axis_index("c")
        sid = lax.axis_index("s")
        wid = sid * 2 + cid

        def start_in(t, dp, dm, sem):
            r0 = (t * NW + wid) * BLK
            pltpu.make_async_copy(p_h.at[pl.ds(r0, BLK), :], dp, sem).start()
            pltpu.make_async_copy(in2_h.at[pl.ds(r0, BLK), :], dm, sem).start()

        def wait_in(dp, dm, sem):
            pltpu.make_async_copy(p_h.at[pl.ds(0, BLK), :], dp, sem).wait()
            pltpu.make_async_copy(in2_h.at[pl.ds(0, BLK), :], dm, sem).wait()

        def start_out(t, src, sem):
            r0 = (t * NW + wid) * BLK
            pltpu.make_async_copy(src, out_h.at[pl.ds(r0, BLK), :], sem).start()

        def wait_out(src, sem):
            pltpu.make_async_copy(src, out_h.at[pl.ds(0, BLK), :], sem).wait()

        start_in(0, pA, mA, sia)

        def pair(i, carry):
            t0 = 2 * i
            wait_in(pA, mA, sia)
            start_in(t0 + 1, pB, mB, sib)

            @pl.when(i > 0)
            def _():
                wait_out(oA, soa)

            _row_compute(pA, mA, oA, BLK)
            start_out(t0, oA, soa)

            wait_in(pB, mB, sib)

            @pl.when(t0 + 2 < T)
            def _():
                start_in(t0 + 2, pA, mA, sia)

            @pl.when(i > 0)
            def _():
                wait_out(oB, sob)

            _row_compute(pB, mB, oB, BLK)
            start_out(t0 + 1, oB, sob)
            return carry

        lax.fori_loop(0, T // 2, pair, 0)
        wait_out(oA, soa)
        wait_out(oB, sob)

        for e in range(EXTRA):
            @pl.when(wid == (e % NW))
            def _():
                r0 = (T * NW + e) * BLK
                pltpu.sync_copy(p_h.at[pl.ds(r0, BLK), :], pA)
                pltpu.sync_copy(in2_h.at[pl.ds(r0, BLK), :], mA)
                _row_compute(pA, mA, oA, BLK)
                pltpu.sync_copy(oA, out_h.at[pl.ds(r0, BLK), :])

    return tec_body


def _build_sc(N):
    assert N % BLK == 0, N
    nblk = N // BLK
    T = (nblk // NW) & ~1
    EXTRA = nblk - T * NW
    f32 = jnp.float32
    mesh = plsc.VectorSubcoreMesh(
        core_axis_name="c", subcore_axis_name="s", num_cores=2, num_subcores=16
    )
    return pl.kernel(
        _make_tec_body(T, EXTRA),
        out_type=jax.ShapeDtypeStruct((N, 32), f32),
        mesh=mesh,
        scratch_types=[
            pltpu.VMEM((BLK, 64), f32),
            pltpu.VMEM((BLK, 1024), f32),
            pltpu.VMEM((BLK, 32), f32),
            pltpu.VMEM((BLK, 64), f32),
            pltpu.VMEM((BLK, 1024), f32),
            pltpu.VMEM((BLK, 32), f32),
            pltpu.SemaphoreType.DMA,
            pltpu.SemaphoreType.DMA,
            pltpu.SemaphoreType.DMA,
            pltpu.SemaphoreType.DMA,
        ],
    )


def kernel(in0, in1, in2):
    p = _p_kernel(in0.T, in1.T)
    return _build_sc(in0.shape[0])(p, in2)
